# baseline (device time: 202824 ns/iter reference)
import jax
import jax.numpy as jnp
from jax import lax
from jax.experimental import pallas as pl
from jax.experimental.pallas import tpu as pltpu

B = 32
H = 16
D = 128
BS = 32
NP = 256
CP = 16
NCH = NP // CP
CK = CP * BS
SCALE = D ** -0.5


def kernel(Q, K, V, bt, lens):
    lens2 = lens.reshape(B, 1)
    Qr = Q.reshape(B, H * D)
    Kr = K.reshape(NP, BS, H * D)
    Vr = V.reshape(NP, BS, H * D)

    def body(q_ref, k_ref, v_ref, bt_ref, lens_ref, out_ref,
             cnt_ref, acc_ref, l_ref, racc_ref, rl_ref,
             send_sems, recv_sems):
        c = pl.program_id(0)
        my_x = lax.axis_index("x")
        my_y = lax.axis_index("y")
        my_z = lax.axis_index("z")
        peer = (my_x, 1 - my_y, my_z)
        bsem = pltpu.get_barrier_semaphore()

        @pl.when(c == 0)
        def _():
            pid = lax.broadcasted_iota(jnp.int32, (B, NP, NP), 1) + my_y * NP
            jmask = (lax.broadcasted_iota(jnp.int32, (B, 1, NP), 2)
                     < lens_ref[...][:, :, None])
            hit = (bt_ref[...][:, None, :] == pid) & jmask
            cnt_ref[...] = jnp.sum(hit.astype(jnp.float32), axis=2)
            acc_ref[...] = jnp.zeros((B, H, D), jnp.float32)
            l_ref[...] = jnp.zeros((B, H), jnp.float32)

        onehot = (lax.broadcasted_iota(jnp.int32, (NP, CK), 0)
                  == c * CP + lax.broadcasted_iota(jnp.int32, (NP, CK), 1) // BS)
        w = lax.dot_general(cnt_ref[...].astype(jnp.bfloat16),
                            onehot.astype(jnp.bfloat16),
                            (((1,), (0,)), ((), ())),
                            preferred_element_type=jnp.float32)

        for h in range(H):
            q_h = q_ref[:, h * D:(h + 1) * D].astype(jnp.bfloat16)
            k_h = k_ref[:, :, h * D:(h + 1) * D].reshape(CK, D).astype(jnp.bfloat16)
            v_h = v_ref[:, :, h * D:(h + 1) * D].reshape(CK, D).astype(jnp.bfloat16)
            s = lax.dot_general(q_h, k_h, (((1,), (1,)), ((), ())),
                                preferred_element_type=jnp.float32) * SCALE
            p = w * jnp.exp(s)
            l_h = jnp.sum(p, axis=1, keepdims=True)
            acc_h = lax.dot_general(p.astype(jnp.bfloat16), v_h,
                                    (((1,), (0,)), ((), ())),
                                    preferred_element_type=jnp.float32)
            acc_ref[:, h, :] = acc_ref[:, h, :] + acc_h
            hcol = lax.broadcasted_iota(jnp.int32, (B, H), 1) == h
            l_ref[...] = l_ref[...] + jnp.where(hcol, l_h, 0.0)

        @pl.when(c == NCH - 1)
        def _():
            pl.semaphore_signal(bsem, inc=1, device_id=peer,
                                device_id_type=pl.DeviceIdType.MESH)
            pl.semaphore_wait(bsem, 1)

            rdma_acc = pltpu.make_async_remote_copy(
                src_ref=acc_ref, dst_ref=racc_ref,
                send_sem=send_sems.at[0], recv_sem=recv_sems.at[0],
                device_id=peer, device_id_type=pl.DeviceIdType.MESH)
            rdma_l = pltpu.make_async_remote_copy(
                src_ref=l_ref, dst_ref=rl_ref,
                send_sem=send_sems.at[1], recv_sem=recv_sems.at[1],
                device_id=peer, device_id_type=pl.DeviceIdType.MESH)
            rdma_acc.start()
            rdma_l.start()
            rdma_acc.wait()
            rdma_l.wait()

            lt = l_ref[...] + rl_ref[...]
            out_ref[...] = (acc_ref[...] + racc_ref[...]) / lt[:, :, None]

    res = pl.pallas_call(
        body,
        grid=(NCH,),
        in_specs=[
            pl.BlockSpec((B, H * D), lambda c: (0, 0)),
            pl.BlockSpec((CP, BS, H * D), lambda c: (c, 0, 0)),
            pl.BlockSpec((CP, BS, H * D), lambda c: (c, 0, 0)),
            pl.BlockSpec((B, NP), lambda c: (0, 0)),
            pl.BlockSpec((B, 1), lambda c: (0, 0)),
        ],
        out_specs=pl.BlockSpec((B, H, D), lambda c: (0, 0, 0)),
        out_shape=jax.ShapeDtypeStruct((B, H, D), jnp.float32),
        scratch_shapes=[
            pltpu.VMEM((B, NP), jnp.float32),
            pltpu.VMEM((B, H, D), jnp.float32),
            pltpu.VMEM((B, H), jnp.float32),
            pltpu.VMEM((B, H, D), jnp.float32),
            pltpu.VMEM((B, H), jnp.float32),
            pltpu.SemaphoreType.DMA((2,)),
            pltpu.SemaphoreType.DMA((2,)),
        ],
        compiler_params=pltpu.CompilerParams(
            dimension_semantics=("arbitrary",),
            collective_id=0,
        ),
    )(Qr, Kr, Vr, bt, lens2)
    return res.reshape(B, 1, H, D)


# device time: 52995 ns/iter; 3.8272x vs baseline; 3.8272x over previous
import jax
import jax.numpy as jnp
from jax import lax
from jax.experimental import pallas as pl
from jax.experimental.pallas import tpu as pltpu

B = 32
H = 16
D = 128
BS = 32
NP = 256
NK = NP * BS
SCALE = D ** -0.5


def kernel(Q, K, V, bt, lens):
    lens2 = lens.reshape(B, 1)

    def body(q_hbm, k_hbm, v_hbm, bt_hbm, lens_hbm, out_ref,
             qv, btv, lensv, w_ref, kbuf, vbuf,
             acc_ref, l_ref, racc_ref, rl_ref,
             insem, ksems, vsems, send_sems, recv_sems):
        my_x = lax.axis_index("x")
        my_y = lax.axis_index("y")
        my_z = lax.axis_index("z")
        peer = (my_x, 1 - my_y, my_z)
        bsem = pltpu.get_barrier_semaphore()

        cp_q = pltpu.make_async_copy(q_hbm, qv, insem.at[0])
        cp_bt = pltpu.make_async_copy(bt_hbm, btv, insem.at[1])
        cp_lens = pltpu.make_async_copy(lens_hbm, lensv, insem.at[2])
        cp_q.start()
        cp_bt.start()
        cp_lens.start()

        def start_head(h):
            slot = h % 3
            pltpu.make_async_copy(
                k_hbm.at[:, :, h, :], kbuf.at[slot], ksems.at[slot]).start()
            pltpu.make_async_copy(
                v_hbm.at[:, :, h, :], vbuf.at[slot], vsems.at[slot]).start()

        def wait_head(h):
            slot = h % 3
            pltpu.make_async_copy(
                k_hbm.at[:, :, h, :], kbuf.at[slot], ksems.at[slot]).wait()
            pltpu.make_async_copy(
                v_hbm.at[:, :, h, :], vbuf.at[slot], vsems.at[slot]).wait()

        start_head(0)
        start_head(1)

        cp_q.wait()
        cp_bt.wait()
        cp_lens.wait()

        pid = lax.broadcasted_iota(jnp.int32, (B, NP, NP), 1) + my_y * NP
        jmask = (lax.broadcasted_iota(jnp.int32, (B, 1, NP), 2)
                 < lensv[...][:, :, None])
        hit = (btv[...][:, None, :] == pid) & jmask
        cnt = jnp.sum(hit.astype(jnp.float32), axis=2)
        onehot = (lax.broadcasted_iota(jnp.int32, (NP, NK), 0)
                  == lax.broadcasted_iota(jnp.int32, (NP, NK), 1) // BS)
        w_ref[...] = lax.dot_general(
            cnt.astype(jnp.bfloat16), onehot.astype(jnp.bfloat16),
            (((1,), (0,)), ((), ())),
            preferred_element_type=jnp.float32)

        for h in range(H):
            wait_head(h)
            if h + 2 < H:
                start_head(h + 2)
            q_h = qv[:, 0, h, :].astype(jnp.bfloat16)
            k2 = kbuf[h % 3].reshape(NK, D).astype(jnp.bfloat16)
            v2 = vbuf[h % 3].reshape(NK, D).astype(jnp.bfloat16)
            s = lax.dot_general(q_h, k2, (((1,), (1,)), ((), ())),
                                preferred_element_type=jnp.float32) * SCALE
            p = w_ref[...] * jnp.exp(s)
            l_h = jnp.sum(p, axis=1, keepdims=True)
            acc_h = lax.dot_general(p.astype(jnp.bfloat16), v2,
                                    (((1,), (0,)), ((), ())),
                                    preferred_element_type=jnp.float32)
            acc_ref[:, h, :] = acc_h
            hcol = lax.broadcasted_iota(jnp.int32, (B, H), 1) == h
            l_ref[...] = jnp.where(hcol, l_h, l_ref[...])

        pl.semaphore_signal(bsem, inc=1, device_id=peer,
                            device_id_type=pl.DeviceIdType.MESH)
        pl.semaphore_wait(bsem, 1)

        rdma_acc = pltpu.make_async_remote_copy(
            src_ref=acc_ref, dst_ref=racc_ref,
            send_sem=send_sems.at[0], recv_sem=recv_sems.at[0],
            device_id=peer, device_id_type=pl.DeviceIdType.MESH)
        rdma_l = pltpu.make_async_remote_copy(
            src_ref=l_ref, dst_ref=rl_ref,
            send_sem=send_sems.at[1], recv_sem=recv_sems.at[1],
            device_id=peer, device_id_type=pl.DeviceIdType.MESH)
        rdma_acc.start()
        rdma_l.start()
        rdma_acc.wait()
        rdma_l.wait()

        lt = l_ref[...] + rl_ref[...]
        out_ref[:, 0, :, :] = (acc_ref[...] + racc_ref[...]) / lt[:, :, None]

    return pl.pallas_call(
        body,
        in_specs=[
            pl.BlockSpec(memory_space=pl.ANY),
            pl.BlockSpec(memory_space=pl.ANY),
            pl.BlockSpec(memory_space=pl.ANY),
            pl.BlockSpec(memory_space=pl.ANY),
            pl.BlockSpec(memory_space=pl.ANY),
        ],
        out_specs=pl.BlockSpec(memory_space=pltpu.VMEM),
        out_shape=jax.ShapeDtypeStruct((B, 1, H, D), jnp.float32),
        scratch_shapes=[
            pltpu.VMEM((B, 1, H, D), jnp.float32),
            pltpu.VMEM((B, NP), jnp.int32),
            pltpu.VMEM((B, 1), jnp.int32),
            pltpu.VMEM((B, NK), jnp.float32),
            pltpu.VMEM((3, NP, BS, D), jnp.float32),
            pltpu.VMEM((3, NP, BS, D), jnp.float32),
            pltpu.VMEM((B, H, D), jnp.float32),
            pltpu.VMEM((B, H), jnp.float32),
            pltpu.VMEM((B, H, D), jnp.float32),
            pltpu.VMEM((B, H), jnp.float32),
            pltpu.SemaphoreType.DMA((3,)),
            pltpu.SemaphoreType.DMA((3,)),
            pltpu.SemaphoreType.DMA((3,)),
            pltpu.SemaphoreType.DMA((2,)),
            pltpu.SemaphoreType.DMA((2,)),
        ],
        compiler_params=pltpu.CompilerParams(
            collective_id=0,
        ),
    )(Q, K, V, bt, lens2)
